# initial kernel scaffold (unmeasured)
import functools

import jax
import jax.numpy as jnp
from jax import lax
from jax.experimental import pallas as pl
from jax.experimental.pallas import tpu as pltpu

N_DEV = 32
BLK = 64


def kernel(x, w_mat):
    k_dim, blk = x.shape
    n = w_mat.shape[1]

    def body(x_ref, w_ref, out_ref, comm_ref, xg_ref, send_sems, recv_sems):
        my_i = lax.axis_index("i")

        barrier_sem = pltpu.get_barrier_semaphore()
        for d in range(1, N_DEV):
            peer = lax.rem(my_i + d, N_DEV)
            pl.semaphore_signal(
                barrier_sem, inc=1,
                device_id=(peer,), device_id_type=pl.DeviceIdType.MESH,
            )
        pl.semaphore_wait(barrier_sem, N_DEV - 1)

        descs = []
        for d in range(1, N_DEV):
            target = lax.rem(my_i + d, N_DEV)
            rdma = pltpu.make_async_remote_copy(
                src_ref=x_ref.at[pl.ds(target * BLK, BLK), :],
                dst_ref=comm_ref.at[d],
                send_sem=send_sems.at[d],
                recv_sem=recv_sems.at[d],
                device_id=(target,),
                device_id_type=pl.DeviceIdType.MESH,
            )
            rdma.start()
            descs.append(rdma)

        xg_ref[:, pl.ds(my_i * BLK, BLK)] = x_ref[pl.ds(my_i * BLK, BLK), :]

        for d in range(1, N_DEV):
            descs[d - 1].wait()
            src_dev = lax.rem(my_i - d + N_DEV, N_DEV)
            xg_ref[:, pl.ds(src_dev * BLK, BLK)] = comm_ref[d]

        y = jnp.dot(xg_ref[...], w_ref[...], preferred_element_type=jnp.float32)
        c = 0.7978845608028654
        out_ref[...] = 0.5 * y * (1.0 + jnp.tanh(c * (y + 0.044715 * y * y * y)))

        @functools.partial(
            pl.run_scoped, exit_sem=pltpu.SemaphoreType.REGULAR
        )
        def _(exit_sem):
            for d in range(1, N_DEV):
                peer = lax.rem(my_i + d, N_DEV)
                pl.semaphore_signal(
                    exit_sem, inc=1,
                    device_id=(peer,), device_id_type=pl.DeviceIdType.MESH,
                )
            pl.semaphore_wait(exit_sem, N_DEV - 1)

    return pl.pallas_call(
        body,
        out_shape=jax.ShapeDtypeStruct((BLK, n), jnp.float32),
        in_specs=[
            pl.BlockSpec(memory_space=pltpu.VMEM),
            pl.BlockSpec(memory_space=pltpu.VMEM),
        ],
        out_specs=pl.BlockSpec(memory_space=pltpu.VMEM),
        scratch_shapes=[
            pltpu.VMEM((N_DEV, BLK, BLK), x.dtype),
            pltpu.VMEM((BLK, k_dim), x.dtype),
            pltpu.SemaphoreType.DMA((N_DEV,)),
            pltpu.SemaphoreType.DMA((N_DEV,)),
        ],
        compiler_params=pltpu.CompilerParams(collective_id=0),
    )(x, w_mat)


# baseline (device time: 37422 ns/iter reference)
import functools

import jax
import jax.numpy as jnp
from jax import lax
from jax.experimental import pallas as pl
from jax.experimental.pallas import tpu as pltpu

N_DEV = 32
BLK = 64


def kernel(x, w_mat):
    k_dim, blk = x.shape
    n = w_mat.shape[1]

    def body(x_ref, w_ref, out_ref, comm_ref, send_sems, recv_sems):
        my_i = lax.axis_index("i")

        barrier_sem = pltpu.get_barrier_semaphore()
        for d in range(1, N_DEV):
            peer = lax.rem(my_i + d, N_DEV)
            pl.semaphore_signal(
                barrier_sem, inc=1,
                device_id=(peer,), device_id_type=pl.DeviceIdType.MESH,
            )
        pl.semaphore_wait(barrier_sem, N_DEV - 1)

        descs = []
        for d in range(1, N_DEV):
            target = lax.rem(my_i + d, N_DEV)
            rdma = pltpu.make_async_remote_copy(
                src_ref=x_ref.at[pl.ds(target * BLK, BLK), :],
                dst_ref=comm_ref.at[pl.ds(my_i * BLK, BLK), :],
                send_sem=send_sems.at[d],
                recv_sem=recv_sems.at[d],
                device_id=(target,),
                device_id_type=pl.DeviceIdType.MESH,
            )
            rdma.start()
            descs.append(rdma)

        comm_ref[pl.ds(my_i * BLK, BLK), :] = x_ref[pl.ds(my_i * BLK, BLK), :]

        for d in range(1, N_DEV):
            descs[d - 1].wait()

        acc = jnp.zeros((BLK, n), dtype=jnp.float32)
        for s in range(N_DEV):
            acc = acc + jnp.dot(
                comm_ref[s * BLK:(s + 1) * BLK, :],
                w_ref[s * BLK:(s + 1) * BLK, :],
                preferred_element_type=jnp.float32,
            )
        c = 0.7978845608028654
        out_ref[...] = 0.5 * acc * (1.0 + jnp.tanh(c * (acc + 0.044715 * acc * acc * acc)))

        @functools.partial(pl.run_scoped, exit_sem=pltpu.SemaphoreType.REGULAR)
        def _(exit_sem):
            for d in range(1, N_DEV):
                peer = lax.rem(my_i + d, N_DEV)
                pl.semaphore_signal(
                    exit_sem, inc=1,
                    device_id=(peer,), device_id_type=pl.DeviceIdType.MESH,
                )
            pl.semaphore_wait(exit_sem, N_DEV - 1)

    return pl.pallas_call(
        body,
        out_shape=jax.ShapeDtypeStruct((BLK, n), jnp.float32),
        in_specs=[
            pl.BlockSpec(memory_space=pltpu.VMEM),
            pl.BlockSpec(memory_space=pltpu.VMEM),
        ],
        out_specs=pl.BlockSpec(memory_space=pltpu.VMEM),
        scratch_shapes=[
            pltpu.VMEM((k_dim, BLK), x.dtype),
            pltpu.SemaphoreType.DMA((N_DEV,)),
            pltpu.SemaphoreType.DMA((N_DEV,)),
        ],
        compiler_params=pltpu.CompilerParams(collective_id=0),
    )(x, w_mat)


# device time: 37024 ns/iter; 1.0107x vs baseline; 1.0107x over previous
import functools

import jax
import jax.numpy as jnp
from jax import lax
from jax.experimental import pallas as pl
from jax.experimental.pallas import tpu as pltpu

N_DEV = 32
BLK = 64


def kernel(x, w_mat):
    k_dim, blk = x.shape
    n = w_mat.shape[1]

    def body(x_ref, w_ref, out_ref, xt_ref, comm_ref, send_sems, recv_sems):
        my_i = lax.axis_index("i")

        barrier_sem = pltpu.get_barrier_semaphore()
        for d in range(1, N_DEV):
            peer = lax.rem(my_i + d, N_DEV)
            pl.semaphore_signal(
                barrier_sem, inc=1,
                device_id=(peer,), device_id_type=pl.DeviceIdType.MESH,
            )
        pl.semaphore_wait(barrier_sem, N_DEV - 1)

        for t in range(N_DEV):
            xt_ref[t * BLK:(t + 1) * BLK, :] = x_ref[
                t * BLK:(t + 1) * BLK, :
            ].T

        descs = []
        for d in range(1, N_DEV):
            target = lax.rem(my_i + d, N_DEV)
            rdma = pltpu.make_async_remote_copy(
                src_ref=xt_ref.at[pl.ds(target * BLK, BLK), :],
                dst_ref=comm_ref.at[pl.ds(my_i * BLK, BLK), :],
                send_sem=send_sems.at[d],
                recv_sem=recv_sems.at[d],
                device_id=(target,),
                device_id_type=pl.DeviceIdType.MESH,
            )
            rdma.start()
            descs.append(rdma)

        comm_ref[pl.ds(my_i * BLK, BLK), :] = xt_ref[pl.ds(my_i * BLK, BLK), :]

        for d in range(1, N_DEV):
            descs[d - 1].wait()

        y = lax.dot_general(
            comm_ref[...], w_ref[...],
            dimension_numbers=(((0,), (0,)), ((), ())),
            preferred_element_type=jnp.float32,
        )
        c = 0.7978845608028654
        out_ref[...] = 0.5 * y * (1.0 + jnp.tanh(c * (y + 0.044715 * y * y * y)))

        @functools.partial(pl.run_scoped, exit_sem=pltpu.SemaphoreType.REGULAR)
        def _(exit_sem):
            for d in range(1, N_DEV):
                peer = lax.rem(my_i + d, N_DEV)
                pl.semaphore_signal(
                    exit_sem, inc=1,
                    device_id=(peer,), device_id_type=pl.DeviceIdType.MESH,
                )
            pl.semaphore_wait(exit_sem, N_DEV - 1)

    return pl.pallas_call(
        body,
        out_shape=jax.ShapeDtypeStruct((BLK, n), jnp.float32),
        in_specs=[
            pl.BlockSpec(memory_space=pltpu.VMEM),
            pl.BlockSpec(memory_space=pltpu.VMEM),
        ],
        out_specs=pl.BlockSpec(memory_space=pltpu.VMEM),
        scratch_shapes=[
            pltpu.VMEM((k_dim, BLK), x.dtype),
            pltpu.VMEM((k_dim, BLK), x.dtype),
            pltpu.SemaphoreType.DMA((N_DEV,)),
            pltpu.SemaphoreType.DMA((N_DEV,)),
        ],
        compiler_params=pltpu.CompilerParams(collective_id=0),
    )(x, w_mat)


# device time: 34891 ns/iter; 1.0725x vs baseline; 1.0611x over previous
import jax
import jax.numpy as jnp
from jax import lax
from jax.experimental import pallas as pl
from jax.experimental.pallas import tpu as pltpu

N_DEV = 32
BLK = 64
N_CHUNKS = 4


def kernel(x, w_mat):
    k_dim, blk = x.shape
    n = w_mat.shape[1]
    nc = n // N_CHUNKS

    def body(x_ref, w_ref, out_ref, xt_ref, comm_ref, wbuf_ref,
             send_sems, recv_sems, w_sems):
        my_i = lax.axis_index("i")

        w_copies = []
        for c in range(N_CHUNKS):
            cp = pltpu.make_async_copy(
                w_ref.at[:, pl.ds(c * nc, nc)],
                wbuf_ref.at[c],
                w_sems.at[c],
            )
            cp.start()
            w_copies.append(cp)

        barrier_sem = pltpu.get_barrier_semaphore()
        for d in range(1, N_DEV):
            peer = lax.rem(my_i + d, N_DEV)
            pl.semaphore_signal(
                barrier_sem, inc=1,
                device_id=(peer,), device_id_type=pl.DeviceIdType.MESH,
            )
        pl.semaphore_wait(barrier_sem, N_DEV - 1)

        for t in range(N_DEV):
            xt_ref[t * BLK:(t + 1) * BLK, :] = x_ref[
                t * BLK:(t + 1) * BLK, :
            ].T

        descs = []
        for d in range(1, N_DEV):
            target = lax.rem(my_i + d, N_DEV)
            rdma = pltpu.make_async_remote_copy(
                src_ref=xt_ref.at[pl.ds(target * BLK, BLK), :],
                dst_ref=comm_ref.at[pl.ds(my_i * BLK, BLK), :],
                send_sem=send_sems.at[d],
                recv_sem=recv_sems.at[d],
                device_id=(target,),
                device_id_type=pl.DeviceIdType.MESH,
            )
            rdma.start()
            descs.append(rdma)

        comm_ref[pl.ds(my_i * BLK, BLK), :] = xt_ref[pl.ds(my_i * BLK, BLK), :]

        for d in range(1, N_DEV):
            descs[d - 1].wait()

        c_gelu = 0.7978845608028654
        for c in range(N_CHUNKS):
            w_copies[c].wait()
            y = lax.dot_general(
                comm_ref[...], wbuf_ref[c],
                dimension_numbers=(((0,), (0,)), ((), ())),
                preferred_element_type=jnp.float32,
            )
            out_ref[:, c * nc:(c + 1) * nc] = 0.5 * y * (
                1.0 + jnp.tanh(c_gelu * (y + 0.044715 * y * y * y))
            )

    return pl.pallas_call(
        body,
        out_shape=jax.ShapeDtypeStruct((BLK, n), jnp.float32),
        in_specs=[
            pl.BlockSpec(memory_space=pltpu.VMEM),
            pl.BlockSpec(memory_space=pl.ANY),
        ],
        out_specs=pl.BlockSpec(memory_space=pltpu.VMEM),
        scratch_shapes=[
            pltpu.VMEM((k_dim, BLK), x.dtype),
            pltpu.VMEM((k_dim, BLK), x.dtype),
            pltpu.VMEM((N_CHUNKS, k_dim, nc), x.dtype),
            pltpu.SemaphoreType.DMA((N_DEV,)),
            pltpu.SemaphoreType.DMA((N_DEV,)),
            pltpu.SemaphoreType.DMA((N_CHUNKS,)),
        ],
        compiler_params=pltpu.CompilerParams(collective_id=0),
    )(x, w_mat)
